# Initial kernel scaffold; baseline (speedup 1.0000x reference)
import jax
import jax.numpy as jnp
from jax.experimental import pallas as pl


def kernel(x, edge_index, batch, W1, a_src1, a_dst1, b1, W2, a_src2, a_dst2, b2, fc1_W, fc1_b, fc2_W, fc2_b):
    raise NotImplementedError("write your pallas kernel here")


# speedup vs baseline: 23.6928x; 23.6928x over previous
import functools

import jax
import jax.numpy as jnp
from jax import lax
from jax.experimental import pallas as pl
from jax.experimental.pallas import tpu as pltpu
from jax.experimental.pallas import tpu_sc as plsc

_N = 10000
_NP = 10240
_D = 128
_H = 128
_NG = 128
_NC = 2
_E = 320000
_EF = _E + _N
_NT = 32
_EPT = 10368
_NCH = 81
_CB = 128
_RPT = _NP // 16

_SELU_SCALE = 1.0507009873554805
_SELU_ALPHA = 1.6732632423543772


def _selu(x):
    return _SELU_SCALE * jnp.where(x > 0, x, _SELU_ALPHA * (jnp.exp(x) - 1.0))


def _dot(a, b):
    return lax.dot_general(a, b, (((1,), (0,)), ((), ())),
                           precision=lax.Precision.HIGHEST,
                           preferred_element_type=jnp.float32)


def _dot00(a, b):
    return lax.dot_general(a, b, (((0,), (0,)), ((), ())),
                           precision=lax.Precision.HIGHEST,
                           preferred_element_type=jnp.float32)


def _proj_body(x_ref, w_ref, asrc_ref, adst_ref, h_ref, al_s_ref, al_d_ref):
    h = _dot(x_ref[...], w_ref[...])
    h_ref[...] = h
    al_s_ref[...] = _dot(h, asrc_ref[...])
    al_d_ref[...] = _dot(h, adst_ref[...])


_proj = pl.pallas_call(
    _proj_body,
    out_shape=(
        jax.ShapeDtypeStruct((_NP, _H), jnp.float32),
        jax.ShapeDtypeStruct((_NP, 1), jnp.float32),
        jax.ShapeDtypeStruct((_NP, 1), jnp.float32),
    ),
)


def _comb_proj_body(acc_ref, sp_ref, b_ref, w_ref, asrc_ref, adst_ref,
                    h2_ref, al_s_ref, al_d_ref):
    ones = jnp.ones((2, 1), jnp.float32)
    s_col = _dot00(sp_ref[...], ones)
    out = (acc_ref[0] + acc_ref[1]) / jnp.maximum(s_col, 1e-30) + b_ref[...]
    hact = _selu(out)
    h2 = _dot(hact, w_ref[...])
    h2_ref[...] = h2
    al_s_ref[...] = _dot(h2, asrc_ref[...])
    al_d_ref[...] = _dot(h2, adst_ref[...])


_comb_proj = pl.pallas_call(
    _comb_proj_body,
    out_shape=(
        jax.ShapeDtypeStruct((_NP, _H), jnp.float32),
        jax.ShapeDtypeStruct((_NP, 1), jnp.float32),
        jax.ShapeDtypeStruct((_NP, 1), jnp.float32),
    ),
)


def _final_body(acc_ref, sp_ref, b_ref, batch_ref, fc1w_ref, fc1b_ref,
                fc2w_ref, fc2b_ref, emb_ref, logp_ref):
    ones = jnp.ones((2, 1), jnp.float32)
    s_col = _dot00(sp_ref[...], ones)
    emb = _selu((acc_ref[0] + acc_ref[1]) / jnp.maximum(s_col, 1e-30) + b_ref[...])
    emb_ref[...] = emb
    gids = lax.broadcasted_iota(jnp.int32, (1, _NG), 1)
    m = jnp.where(batch_ref[...] == gids, 1.0, 0.0)
    pooled_sum = _dot00(m, emb)
    counts = _dot00(m, jnp.ones((_NP, 1), jnp.float32))
    pooled = pooled_sum / jnp.maximum(counts, 1.0)
    g = _selu(pooled)
    g = _selu(_dot(g, fc1w_ref[...]) + fc1b_ref[...])
    logits = _dot(g, fc2w_ref[...]) + fc2b_ref[...]
    mx = jnp.max(logits, axis=1, keepdims=True)
    l = logits - mx
    lse = jnp.log(jnp.sum(jnp.exp(l), axis=1, keepdims=True))
    logp_ref[...] = l - lse


_final = pl.pallas_call(
    _final_body,
    out_shape=(
        jax.ShapeDtypeStruct((_NP, _H), jnp.float32),
        jax.ShapeDtypeStruct((_NG, _NC), jnp.float32),
    ),
)


def _edge_body(h_hbm, als_hbm, ald_hbm, src_hbm, dst_hbm,
               acc_hbm, sp_hbm,
               src_v, dst_v, rows_v, eas_v, ead_v, ee_v, zs_v, acc_sh, s_sh,
               sem_s, sem_d, sem_r):
    cid = lax.axis_index("c")
    sid = lax.axis_index("s")
    wid = sid * 2 + cid

    pltpu.sync_copy(src_hbm.at[wid], src_v)
    pltpu.sync_copy(dst_hbm.at[wid], dst_v)

    zero16 = jnp.zeros((16,), jnp.float32)

    def zrow(i, c):
        for mi in range(8):
            rows_v[i, pl.ds(mi * 16, 16)] = zero16
        return c
    lax.fori_loop(0, _CB, zrow, 0)

    def zs(i, c):
        zs_v[pl.ds(i * 16, 16)] = zero16
        return c
    lax.fori_loop(0, _RPT // 16, zs, 0)

    base = sid * _RPT
    for b in range(_RPT // _CB):
        pltpu.sync_copy(rows_v, acc_sh.at[pl.ds(base + b * _CB, _CB)])
    pltpu.sync_copy(zs_v, s_sh.at[pl.ds(base, _RPT)])
    plsc.subcore_barrier()

    ebase = wid * _EPT

    def p2(j, c):
        cs = pltpu.async_copy(als_hbm.at[src_v.at[j]], eas_v, sem_s)
        cd = pltpu.async_copy(ald_hbm.at[dst_v.at[j]], ead_v, sem_d)
        cr = pltpu.async_copy(h_hbm.at[src_v.at[j]], rows_v, sem_r)
        cs.wait()
        cd.wait()
        gbase = ebase + j * _CB

        for k in range(_CB // 16):
            a1 = eas_v[pl.ds(k * 16, 16)]
            a2 = ead_v[pl.ds(k * 16, 16)]
            t = a1 + a2
            e = jnp.where(t >= 0, t, 0.2 * t)
            ee = jnp.exp(e)
            gid = gbase + k * 16 + lax.iota(jnp.int32, 16)
            ee = jnp.where(gid < _EF, ee, 0.0)
            ee_v[pl.ds(k * 16, 16)] = ee

        cr.wait()

        def scale(e, c2):
            e16 = jnp.full((16,), e, jnp.int32)
            eev = plsc.load_gather(ee_v, [e16])
            for mi in range(8):
                sl = pl.ds(mi * 16, 16)
                rows_v[e, sl] = rows_v[e, sl] * eev
            return c2
        lax.fori_loop(0, _CB, scale, 0)

        pltpu.sync_copy(rows_v, acc_sh.at[dst_v.at[j]], add=True)
        pltpu.sync_copy(ee_v, s_sh.at[dst_v.at[j]], add=True)
        return c
    lax.fori_loop(0, _NCH, p2, 0)

    plsc.subcore_barrier()
    for b in range(_RPT // _CB):
        sl = pl.ds(base + b * _CB, _CB)
        pltpu.sync_copy(acc_sh.at[sl], acc_hbm.at[cid, sl])
    pltpu.sync_copy(s_sh.at[pl.ds(base, _RPT)], sp_hbm.at[cid, pl.ds(base, _RPT)])


_edge = functools.partial(
    pl.kernel,
    out_type=(
        jax.ShapeDtypeStruct((2, _NP, _H), jnp.float32),
        jax.ShapeDtypeStruct((2, _NP), jnp.float32),
    ),
    mesh=plsc.VectorSubcoreMesh(core_axis_name="c", subcore_axis_name="s"),
    compiler_params=pltpu.CompilerParams(needs_layout_passes=False),
    scratch_types=(
        pltpu.VMEM((_NCH, _CB), jnp.int32),
        pltpu.VMEM((_NCH, _CB), jnp.int32),
        pltpu.VMEM((_CB, _H), jnp.float32),
        pltpu.VMEM((_CB,), jnp.float32),
        pltpu.VMEM((_CB,), jnp.float32),
        pltpu.VMEM((_CB,), jnp.float32),
        pltpu.VMEM((_RPT,), jnp.float32),
        pltpu.VMEM_SHARED((_NP, _H), jnp.float32),
        pltpu.VMEM_SHARED((_NP,), jnp.float32),
        pltpu.SemaphoreType.DMA,
        pltpu.SemaphoreType.DMA,
        pltpu.SemaphoreType.DMA,
    ),
)(_edge_body)


def kernel(x, edge_index, batch, W1, a_src1, a_dst1, b1,
           W2, a_src2, a_dst2, b2, fc1_W, fc1_b, fc2_W, fc2_b):
    loops = jnp.arange(_N, dtype=jnp.int32)
    pad = jnp.zeros((_NT * _EPT - _EF,), jnp.int32)
    src = jnp.concatenate([edge_index[0], loops, pad]).reshape(_NT, _NCH, _CB)
    dst = jnp.concatenate([edge_index[1], loops, pad]).reshape(_NT, _NCH, _CB)

    xp = jnp.pad(x, ((0, _NP - _N), (0, 0)))
    batch_p = jnp.pad(batch.astype(jnp.int32), (0, _NP - _N),
                      constant_values=_NG)
    h1, als1, ald1 = _proj(xp, W1, a_src1.reshape(_H, 1), a_dst1.reshape(_H, 1))
    acc1, sp1 = _edge(h1, als1.reshape(_NP), ald1.reshape(_NP), src, dst)
    h2, als2, ald2 = _comb_proj(acc1, sp1, b1.reshape(1, _H), W2,
                                a_src2.reshape(_H, 1), a_dst2.reshape(_H, 1))
    acc2, sp2 = _edge(h2, als2.reshape(_NP), ald2.reshape(_NP), src, dst)
    emb, logp = _final(acc2, sp2, b2.reshape(1, _H), batch_p.reshape(_NP, 1),
                       fc1_W, fc1_b.reshape(1, -1), fc2_W, fc2_b.reshape(1, -1))
    return (emb[:_N], logp)
